# Initial kernel scaffold; baseline (speedup 1.0000x reference)
import jax
import jax.numpy as jnp
from jax.experimental import pallas as pl


def kernel(x, table):
    raise NotImplementedError("write your pallas kernel here")


# speedup vs baseline: 4.9783x; 4.9783x over previous
import functools

import jax
import jax.numpy as jnp
from jax import lax
from jax.experimental import pallas as pl
from jax.experimental.pallas import tpu as pltpu
from jax.experimental.pallas import tpu_sc as plsc

EMBEDDING_DIM = 32

_info = plsc.get_sparse_core_info()
_NC, _NS = _info.num_cores, _info.num_subcores
_NW = _NC * _NS

_CHUNK = 2048


def _make_gather(B: int, D: int):
    assert B % (_NW * _CHUNK) == 0
    b_per_w = B // _NW
    n_chunks = b_per_w // _CHUNK
    mesh = plsc.VectorSubcoreMesh(core_axis_name="c", subcore_axis_name="s")

    @functools.partial(
        pl.kernel,
        out_type=jax.ShapeDtypeStruct((B, D), jnp.float32),
        mesh=mesh,
        scratch_types=[
            pltpu.VMEM((_CHUNK,), jnp.int32),
            pltpu.VMEM((_CHUNK, D), jnp.float32),
            pltpu.SemaphoreType.DMA,
        ],
        compiler_params=pltpu.CompilerParams(use_tc_tiling_on_sc=False),
    )
    def gather_kernel(table_hbm, idx_hbm, out_hbm, idx_v, rows_v, sem):
        wid = lax.axis_index("s") * _NC + lax.axis_index("c")
        base = wid * b_per_w

        def body(i, carry):
            off = base + i * _CHUNK
            pltpu.sync_copy(idx_hbm.at[pl.ds(off, _CHUNK)], idx_v)
            pltpu.async_copy(table_hbm.at[idx_v], rows_v, sem).wait()
            pltpu.sync_copy(rows_v, out_hbm.at[pl.ds(off, _CHUNK)])
            return carry

        lax.fori_loop(0, n_chunks, body, 0, unroll=False)

    return gather_kernel


def kernel(x, table):
    B = x.size
    idx = x.reshape(-1).astype(jnp.int32)
    out = _make_gather(B, EMBEDDING_DIM)(table, idx)
    return out.reshape(x.shape + (EMBEDDING_DIM,))


# speedup vs baseline: 6.7727x; 1.3604x over previous
import functools

import jax
import jax.numpy as jnp
from jax import lax
from jax.experimental import pallas as pl
from jax.experimental.pallas import tpu as pltpu
from jax.experimental.pallas import tpu_sc as plsc

EMBEDDING_DIM = 32

_info = plsc.get_sparse_core_info()
_NC, _NS = _info.num_cores, _info.num_subcores
_NW = _NC * _NS
_L = 16

_CHUNK = 1024


def _make_gather(B: int, NB: int, D: int):
    assert B % (_NW * _CHUNK) == 0 and NB % _CHUNK == 0
    n_chunks = (B // _NW) // _CHUNK
    chunks_per_row = NB // _CHUNK
    mesh = plsc.VectorSubcoreMesh(core_axis_name="c", subcore_axis_name="s")

    @functools.partial(
        pl.kernel,
        out_type=jax.ShapeDtypeStruct((B // NB * D, NB), jnp.float32),
        mesh=mesh,
        scratch_types=[
            pltpu.VMEM((_CHUNK,), jnp.int32),
            pltpu.VMEM((_CHUNK,), jnp.int32),
            pltpu.VMEM((_CHUNK, D), jnp.float32),
            pltpu.VMEM((_CHUNK, D), jnp.float32),
            pltpu.VMEM((D, _CHUNK), jnp.float32),
            pltpu.SemaphoreType.DMA,
            pltpu.SemaphoreType.DMA,
            pltpu.SemaphoreType.DMA,
        ],
        compiler_params=pltpu.CompilerParams(
            use_tc_tiling_on_sc=False, needs_layout_passes=False
        ),
    )
    def gather_kernel(table_hbm, idx_hbm, out_hbm, idx0, idx1, rows0, rows1,
                      rows_t, gs0, gs1, ws):
        wid = lax.axis_index("s") * _NC + lax.axis_index("c")
        g_base = wid * n_chunks
        idx_b = (idx0, idx1)
        rows_b = (rows0, rows1)
        gs = (gs0, gs1)
        lanes = jnp.arange(_L, dtype=jnp.int32)

        def fire(g, b):
            pltpu.sync_copy(idx_hbm.at[pl.ds(g * _CHUNK, _CHUNK)], idx_b[b])
            pltpu.async_copy(table_hbm.at[idx_b[b]], rows_b[b], gs[b])

        def drain(g, b, not_first):
            pltpu.make_async_copy(table_hbm.at[idx_b[b]], rows_b[b], gs[b]).wait()

            @pl.when(not_first)
            def _():
                pltpu.make_async_copy(
                    rows_t, out_hbm.at[pl.ds(0, D), pl.ds(0, _CHUNK)], ws
                ).wait()

            rows = rows_b[b]

            def jb_body(jb, carry):
                j0v = lanes + jb * _L
                r = lanes
                for _k in range(_L):
                    for d0 in (0, _L):
                        c = r + d0 if d0 else r
                        v = plsc.load_gather(rows, [j0v, c])
                        plsc.store_scatter(rows_t, [c, j0v], v)
                    r = (r + 1) & (_L - 1)
                return carry

            lax.fori_loop(0, _CHUNK // _L, jb_body, 0, unroll=False)

            t = g // chunks_per_row
            b0 = (g % chunks_per_row) * _CHUNK
            pltpu.async_copy(
                rows_t, out_hbm.at[pl.ds(t * D, D), pl.ds(b0, _CHUNK)], ws
            )

        fire(g_base, 0)

        def body(p, carry):
            g0 = g_base + 2 * p
            fire(g0 + 1, 1)
            drain(g0, 0, p > 0)

            @pl.when(2 * p + 2 < n_chunks)
            def _():
                fire(g0 + 2, 0)

            drain(g0 + 1, 1, p >= 0)
            return carry

        lax.fori_loop(0, n_chunks // 2, body, 0, unroll=False)
        pltpu.make_async_copy(
            rows_t, out_hbm.at[pl.ds(0, D), pl.ds(0, _CHUNK)], ws
        ).wait()

    return gather_kernel


def kernel(x, table):
    NB, NT = x.shape
    B = NB * NT
    idx = x.T.reshape(-1).astype(jnp.int32)
    out2d = _make_gather(B, NB, EMBEDDING_DIM)(table, idx)
    out = out2d.reshape(NT, EMBEDDING_DIM, NB).transpose(2, 0, 1)
    return out
